# baseline (speedup 1.0000x reference)
import jax
import jax.numpy as jnp
from jax import lax
from jax.experimental import pallas as pl
from jax.experimental.pallas import tpu as pltpu
from jax.experimental.pallas import tpu_sc as plsc

_VOCAB = 1000
_EMB = 64
_BATCH = 16384
_HIST = 50

_NC = 2
_NS = 16
_NW = _NC * _NS

_ROWS_W = _BATCH // _NW
_CB = 4
_NCHUNKS = _ROWS_W // _CB
_NBUF = 4


def _body(ids_hbm, table_hbm, out_hbm, idx_v, buf0, buf1, buf2, buf3,
          gsem0, gsem1, gsem2, gsem3, ssem0, ssem1, ssem2, ssem3):
    c_id = lax.axis_index("c")
    s_id = lax.axis_index("s")
    wid = s_id * _NC + c_id
    base = wid * _ROWS_W

    pltpu.sync_copy(ids_hbm.at[pl.ds(base, _ROWS_W)], idx_v)

    bufs = (buf0, buf1, buf2, buf3)
    gsems = (gsem0, gsem1, gsem2, gsem3)
    ssems = (ssem0, ssem1, ssem2, ssem3)

    def gather_descs(c, b):
        return [
            pltpu.make_async_copy(
                table_hbm.at[idx_v.at[c * _CB + j]],
                bufs[b].at[j],
                gsems[b],
            )
            for j in range(_CB)
        ]

    def store_desc(c, b):
        return pltpu.make_async_copy(
            bufs[b], out_hbm.at[pl.ds(base + c * _CB, _CB)], ssems[b]
        )

    def fire_gathers(c, b):
        for d in gather_descs(c, b):
            d.start()

    fire_gathers(0, 0)

    def loop_body(i, carry):
        for b in range(_NBUF):
            c = _NBUF * i + b
            nb = (b + 1) % _NBUF

            @pl.when(c >= _NBUF - 1)
            def _():
                store_desc(c - (_NBUF - 1), nb).wait()

            @pl.when(c + 1 < _NCHUNKS)
            def _():
                fire_gathers(c + 1, nb)

            for d in gather_descs(c, b):
                d.wait()
            store_desc(c, b).start()
        return carry

    lax.fori_loop(0, _NCHUNKS // _NBUF, loop_body, 0)

    for c in range(_NCHUNKS - (_NBUF - 1), _NCHUNKS):
        store_desc(c, c % _NBUF).wait()


def kernel(ids, table):
    run = pl.kernel(
        _body,
        out_type=jax.ShapeDtypeStruct((_BATCH, _HIST, _EMB), jnp.float32),
        mesh=plsc.VectorSubcoreMesh(core_axis_name="c", subcore_axis_name="s"),
        compiler_params=pltpu.CompilerParams(use_tc_tiling_on_sc=False),
        scratch_types=[
            pltpu.VMEM((_ROWS_W, _HIST), jnp.int32),
            pltpu.VMEM((_CB, _HIST, _EMB), jnp.float32),
            pltpu.VMEM((_CB, _HIST, _EMB), jnp.float32),
            pltpu.VMEM((_CB, _HIST, _EMB), jnp.float32),
            pltpu.VMEM((_CB, _HIST, _EMB), jnp.float32),
            pltpu.SemaphoreType.DMA,
            pltpu.SemaphoreType.DMA,
            pltpu.SemaphoreType.DMA,
            pltpu.SemaphoreType.DMA,
            pltpu.SemaphoreType.DMA,
            pltpu.SemaphoreType.DMA,
            pltpu.SemaphoreType.DMA,
            pltpu.SemaphoreType.DMA,
        ],
    )
    return run(ids, table)
